# baseline (device time: 224562 ns/iter reference)
import jax
import jax.numpy as jnp
from jax import lax
from jax.experimental import pallas as pl
from jax.experimental.pallas import tpu as pltpu

N_DEV = 8
NHOP = N_DEV - 1
B = 2
SQ = 512
DM = 768
HQ = 8
DH = 64
DQ = HQ * DH
SKV_LOC = 512
SLOT_ROWS = SQ + 2 * HQ


def kernel(x, Wq, K_ext, V_ext, Wo):
    def body(x_ref, wq_ref, k_ref, v_ref, wo_ref, out_ref,
             comm_ref, send_sems, recv_sems):
        my = lax.axis_index("i")
        left = lax.rem(my + N_DEV - 1, N_DEV)
        right = lax.rem(my + 1, N_DEV)

        barrier_sem = pltpu.get_barrier_semaphore()
        for nbr in (left, right):
            pl.semaphore_signal(
                barrier_sem, inc=1,
                device_id=(nbr,), device_id_type=pl.DeviceIdType.MESH,
            )
        pl.semaphore_wait(barrier_sem, 2)

        ri = lax.broadcasted_iota(jnp.int32, (SQ, SKV_LOC), 0)
        rj = lax.broadcasted_iota(jnp.int32, (SQ, SKV_LOC), 1)
        mask = ((ri // 64) % 4) == ((rj // 64) % 4)

        for b in range(B):
            q_b = lax.dot(x_ref[b], wq_ref[...],
                          preferred_element_type=jnp.float32)
            for h in range(HQ):
                q_bh = q_b[:, h * DH:(h + 1) * DH]
                k_bh = k_ref[b, :, h, :]
                v_bh = v_ref[b, :, h, :]
                s = lax.dot_general(
                    q_bh, k_bh, (((1,), (1,)), ((), ())),
                    preferred_element_type=jnp.float32,
                ) * 0.125
                s = jnp.where(mask, s, -1e9)
                m_bh = jnp.max(s, axis=1)
                w = jnp.exp(s - m_bh[:, None])
                l_bh = jnp.sum(w, axis=1)
                acc_bh = lax.dot_general(
                    w, v_bh, (((1,), (0,)), ((), ())),
                    preferred_element_type=jnp.float32,
                )
                comm_ref[0, b, :SQ, h * DH:(h + 1) * DH] = acc_bh
                comm_ref[0, b, SQ + h, :] = m_bh
                comm_ref[0, b, SQ + HQ + h, :] = l_bh

        for h in range(NHOP):
            rdma = pltpu.make_async_remote_copy(
                src_ref=comm_ref.at[h],
                dst_ref=comm_ref.at[h + 1],
                send_sem=send_sems.at[h],
                recv_sem=recv_sems.at[h],
                device_id=(right,),
                device_id_type=pl.DeviceIdType.MESH,
            )
            rdma.start()
            rdma.wait()

        for b in range(B):
            m_all = [comm_ref[s, b, SQ:SQ + HQ, :] for s in range(N_DEV)]
            l_all = [comm_ref[s, b, SQ + HQ:SQ + 2 * HQ, :] for s in range(N_DEV)]
            m_g = m_all[0]
            for s in range(1, N_DEV):
                m_g = jnp.maximum(m_g, m_all[s])
            c_all = [jnp.exp(m_all[s] - m_g) for s in range(N_DEV)]
            l_tot = c_all[0] * l_all[0]
            for s in range(1, N_DEV):
                l_tot = l_tot + c_all[s] * l_all[s]
            inv = 1.0 / l_tot
            ctx_cols = []
            for h in range(HQ):
                num = comm_ref[0, b, :SQ, h * DH:(h + 1) * DH] * c_all[0][h][:, None]
                for s in range(1, N_DEV):
                    num = num + (comm_ref[s, b, :SQ, h * DH:(h + 1) * DH]
                                 * c_all[s][h][:, None])
                ctx_cols.append(num * inv[h][:, None])
            ctx_b = jnp.concatenate(ctx_cols, axis=1)
            out_ref[b] = lax.dot(ctx_b, wo_ref[...],
                                 preferred_element_type=jnp.float32)

    return pl.pallas_call(
        body,
        out_shape=jax.ShapeDtypeStruct((B, SQ, DM), jnp.float32),
        in_specs=[pl.BlockSpec(memory_space=pltpu.VMEM)] * 5,
        out_specs=pl.BlockSpec(memory_space=pltpu.VMEM),
        scratch_shapes=[
            pltpu.VMEM((N_DEV, B, SLOT_ROWS, DQ), jnp.float32),
            pltpu.SemaphoreType.DMA((NHOP,)),
            pltpu.SemaphoreType.DMA((NHOP,)),
        ],
        compiler_params=pltpu.CompilerParams(collective_id=0),
    )(x, Wq, K_ext, V_ext, Wo)


# device time: 60593 ns/iter; 3.7061x vs baseline; 3.7061x over previous
import jax
import jax.numpy as jnp
from jax import lax
from jax.experimental import pallas as pl
from jax.experimental.pallas import tpu as pltpu

N_DEV = 8
B = 2
SQ = 512
DM = 768
HQ = 8
DH = 64
DQ = HQ * DH
SKV_LOC = 512
CH = SQ // N_DEV


def kernel(x, Wq, K_ext, V_ext, Wo):
    def body(x_ref, wq_ref, k_ref, v_ref, wo_ref, out_ref,
             accc_ref,
             stat_ref,
             racc_ref,
             rstat_ref,
             ag_ref,
             rs_acc_send_sems, rs_stat_send_sems,
             rs_acc_recv_sems, rs_stat_recv_sems,
             ag_send_sems, ag_recv_sems):
        me = lax.axis_index("i")

        barrier_sem = pltpu.get_barrier_semaphore()
        for d in range(N_DEV):
            @pl.when(d != me)
            def _():
                pl.semaphore_signal(
                    barrier_sem, inc=1,
                    device_id=(d,), device_id_type=pl.DeviceIdType.MESH,
                )
        pl.semaphore_wait(barrier_sem, N_DEV - 1)

        ri = lax.broadcasted_iota(jnp.int32, (SQ, SKV_LOC), 0)
        rj = lax.broadcasted_iota(jnp.int32, (SQ, SKV_LOC), 1)
        mask = ((ri // 64) % 4) == ((rj // 64) % 4)

        for b in range(B):
            q_b = lax.dot(x_ref[b], wq_ref[...],
                          preferred_element_type=jnp.float32)
            for h in range(HQ):
                q_bh = q_b[:, h * DH:(h + 1) * DH]
                k_bh = k_ref[b, :, h, :]
                v_bh = v_ref[b, :, h, :]
                s = lax.dot_general(
                    q_bh, k_bh, (((1,), (1,)), ((), ())),
                    preferred_element_type=jnp.float32,
                ) * 0.125
                s = jnp.where(mask, s, -1e9)
                m_bh = jnp.max(s, axis=1)
                w = jnp.exp(s - m_bh[:, None])
                l_bh = jnp.sum(w, axis=1)
                acc_bh = lax.dot_general(
                    w, v_bh, (((1,), (0,)), ((), ())),
                    preferred_element_type=jnp.float32,
                )
                for c in range(N_DEV):
                    rows = slice(c * CH, (c + 1) * CH)
                    accc_ref[c, b, :, h * DH:(h + 1) * DH] = acc_bh[rows, :]
                    stat_ref[c, b, h, :] = m_bh[rows]
                    stat_ref[c, b, HQ + h, :] = l_bh[rows]

        rs_sends = []
        for c in range(N_DEV):
            slot = lax.rem(me - c - 1 + 2 * N_DEV, N_DEV)
            acc_rdma = pltpu.make_async_remote_copy(
                src_ref=accc_ref.at[c],
                dst_ref=racc_ref.at[slot],
                send_sem=rs_acc_send_sems.at[c],
                recv_sem=rs_acc_recv_sems.at[slot],
                device_id=(c,), device_id_type=pl.DeviceIdType.MESH,
            )
            stat_rdma = pltpu.make_async_remote_copy(
                src_ref=stat_ref.at[c],
                dst_ref=rstat_ref.at[slot],
                send_sem=rs_stat_send_sems.at[c],
                recv_sem=rs_stat_recv_sems.at[slot],
                device_id=(c,), device_id_type=pl.DeviceIdType.MESH,
            )

            @pl.when(c != me)
            def _():
                acc_rdma.start()
                stat_rdma.start()

            rs_sends.append((c, acc_rdma, stat_rdma))

        for k in range(N_DEV - 1):
            recv_acc = pltpu.make_async_remote_copy(
                src_ref=accc_ref.at[0], dst_ref=racc_ref.at[k],
                send_sem=rs_acc_send_sems.at[0],
                recv_sem=rs_acc_recv_sems.at[k],
                device_id=(me,), device_id_type=pl.DeviceIdType.MESH,
            )
            recv_stat = pltpu.make_async_remote_copy(
                src_ref=stat_ref.at[0], dst_ref=rstat_ref.at[k],
                send_sem=rs_stat_send_sems.at[0],
                recv_sem=rs_stat_recv_sems.at[k],
                device_id=(me,), device_id_type=pl.DeviceIdType.MESH,
            )
            recv_acc.wait_recv()
            recv_stat.wait_recv()

        m_srcs = [stat_ref[me, :, 0:HQ, :]] + [
            rstat_ref[k, :, 0:HQ, :] for k in range(N_DEV - 1)
        ]
        l_srcs = [stat_ref[me, :, HQ:2 * HQ, :]] + [
            rstat_ref[k, :, HQ:2 * HQ, :] for k in range(N_DEV - 1)
        ]
        acc_srcs = [accc_ref[me]] + [racc_ref[k] for k in range(N_DEV - 1)]

        m_g = m_srcs[0]
        for s_ in m_srcs[1:]:
            m_g = jnp.maximum(m_g, s_)
        e_srcs = [jnp.exp(m_ - m_g) for m_ in m_srcs]
        l_tot = e_srcs[0] * l_srcs[0]
        for e_, l_ in zip(e_srcs[1:], l_srcs[1:]):
            l_tot = l_tot + e_ * l_
        inv_l = 1.0 / l_tot

        for h in range(HQ):
            cols = slice(h * DH, (h + 1) * DH)
            num = acc_srcs[0][:, :, cols] * e_srcs[0][:, h, :][:, :, None]
            for s_ in range(1, N_DEV):
                num = num + acc_srcs[s_][:, :, cols] * e_srcs[s_][:, h, :][:, :, None]
            ag_ref[me, :, :, cols] = num * inv_l[:, h, :][:, :, None]

        ag_sends = []
        for t in range(1, N_DEV):
            tgt = lax.rem(me + t, N_DEV)
            rdma = pltpu.make_async_remote_copy(
                src_ref=ag_ref.at[me],
                dst_ref=ag_ref.at[me],
                send_sem=ag_send_sems.at[t - 1],
                recv_sem=ag_recv_sems.at[N_DEV - 1 - t],
                device_id=(tgt,), device_id_type=pl.DeviceIdType.MESH,
            )
            rdma.start()
            ag_sends.append(rdma)

        for b in range(B):
            out_ref[b, pl.ds(me * CH, CH), :] = lax.dot(
                ag_ref[me, b], wo_ref[...], preferred_element_type=jnp.float32)

        for k in range(N_DEV - 1):
            recv = pltpu.make_async_remote_copy(
                src_ref=ag_ref.at[0], dst_ref=ag_ref.at[me],
                send_sem=ag_send_sems.at[0],
                recv_sem=ag_recv_sems.at[k],
                device_id=(me,), device_id_type=pl.DeviceIdType.MESH,
            )
            recv.wait_recv()
            src = lax.rem(me + k + 1, N_DEV)
            for b in range(B):
                out_ref[b, pl.ds(src * CH, CH), :] = lax.dot(
                    ag_ref[src, b], wo_ref[...],
                    preferred_element_type=jnp.float32)

        for c, acc_rdma, stat_rdma in rs_sends:
            @pl.when(c != me)
            def _():
                acc_rdma.wait_send()
                stat_rdma.wait_send()
        for rdma in ag_sends:
            rdma.wait_send()

    return pl.pallas_call(
        body,
        out_shape=jax.ShapeDtypeStruct((B, SQ, DM), jnp.float32),
        in_specs=[pl.BlockSpec(memory_space=pltpu.VMEM)] * 5,
        out_specs=pl.BlockSpec(memory_space=pltpu.VMEM),
        scratch_shapes=[
            pltpu.VMEM((N_DEV, B, CH, DQ), jnp.float32),
            pltpu.VMEM((N_DEV, B, 2 * HQ, CH), jnp.float32),
            pltpu.VMEM((N_DEV - 1, B, CH, DQ), jnp.float32),
            pltpu.VMEM((N_DEV - 1, B, 2 * HQ, CH), jnp.float32),
            pltpu.VMEM((N_DEV, B, CH, DQ), jnp.float32),
            pltpu.SemaphoreType.DMA((N_DEV,)),
            pltpu.SemaphoreType.DMA((N_DEV,)),
            pltpu.SemaphoreType.DMA((N_DEV - 1,)),
            pltpu.SemaphoreType.DMA((N_DEV - 1,)),
            pltpu.SemaphoreType.DMA((N_DEV - 1,)),
            pltpu.SemaphoreType.DMA((N_DEV - 1,)),
        ],
        compiler_params=pltpu.CompilerParams(collective_id=0),
    )(x, Wq, K_ext, V_ext, Wo)


# device time: 47277 ns/iter; 4.7499x vs baseline; 1.2817x over previous
import jax
import jax.numpy as jnp
from jax import lax
from jax.experimental import pallas as pl
from jax.experimental.pallas import tpu as pltpu

N_DEV = 8
B = 2
SQ = 512
DM = 768
HQ = 8
DH = 64
DQ = HQ * DH
CH = SQ // N_DEV
WIRE = jnp.bfloat16
PROWS = CH + 2 * HQ


def kernel(x, Wq, K_ext, V_ext, Wo):
    def body(x_ref, wq_ref, k_ref, v_ref, wo_ref, out_ref,
             part_ref,
             rrecv_ref,
             ag_ref,
             rs_send_sems,
             rs_recv_sems,
             ag_send_sems, ag_recv_sems):
        me = lax.axis_index("i")

        barrier_sem = pltpu.get_barrier_semaphore()
        for d in range(N_DEV):
            @pl.when(d != me)
            def _():
                pl.semaphore_signal(
                    barrier_sem, inc=1,
                    device_id=(d,), device_id_type=pl.DeviceIdType.MESH,
                )
        pl.semaphore_wait(barrier_sem, N_DEV - 1)

        rs_sends = []
        for b in range(B):
            q_b = lax.dot(x_ref[b], wq_ref[...],
                          preferred_element_type=jnp.float32)
            for h in range(HQ):
                cols = slice(h * DH, (h + 1) * DH)
                for r in range(4):
                    lo = slice(r * 64, r * 64 + 64)
                    hi = slice((r + 4) * 64, (r + 4) * 64 + 64)
                    q_r = jnp.concatenate(
                        [q_b[lo, cols], q_b[hi, cols]], axis=0)
                    k_r = jnp.concatenate(
                        [k_ref[b, lo, h, :], k_ref[b, hi, h, :]], axis=0)
                    v_r = jnp.concatenate(
                        [v_ref[b, lo, h, :], v_ref[b, hi, h, :]], axis=0)
                    s = lax.dot_general(
                        q_r, k_r, (((1,), (1,)), ((), ())),
                        preferred_element_type=jnp.float32,
                    ) * 0.125
                    m_r = jnp.max(s, axis=1)
                    w = jnp.exp(s - m_r[:, None])
                    l_r = jnp.sum(w, axis=1)
                    acc_r = lax.dot_general(
                        w, v_r, (((1,), (0,)), ((), ())),
                        preferred_element_type=jnp.float32,
                    ).astype(WIRE)
                    for half, qb in ((0, r), (1, r + 4)):
                        rows = slice(half * 64, half * 64 + 64)
                        part_ref[qb, b, :CH, cols] = acc_r[rows, :]
                        part_ref[qb, b, CH + h, :CH] = m_r[rows].astype(WIRE)
                        part_ref[qb, b, CH + HQ + h, :CH] = l_r[rows].astype(WIRE)
            for c in range(N_DEV):
                slot = lax.rem(me - c - 1 + 2 * N_DEV, N_DEV)
                rdma = pltpu.make_async_remote_copy(
                    src_ref=part_ref.at[c, b],
                    dst_ref=rrecv_ref.at[slot, b],
                    send_sem=rs_send_sems.at[b, c],
                    recv_sem=rs_recv_sems.at[b, slot],
                    device_id=(c,), device_id_type=pl.DeviceIdType.MESH,
                )

                @pl.when(c != me)
                def _():
                    rdma.start()

                rs_sends.append((c, rdma))

        for b in range(B):
            for k in range(N_DEV - 1):
                recv = pltpu.make_async_remote_copy(
                    src_ref=part_ref.at[0, b], dst_ref=rrecv_ref.at[k, b],
                    send_sem=rs_send_sems.at[b, 0],
                    recv_sem=rs_recv_sems.at[b, k],
                    device_id=(me,), device_id_type=pl.DeviceIdType.MESH,
                )
                recv.wait_recv()

        def stats(ref, idx):
            m_ = ref[idx, :, CH:CH + HQ, :CH].astype(jnp.float32)
            l_ = ref[idx, :, CH + HQ:CH + 2 * HQ, :CH].astype(jnp.float32)
            return m_, l_

        srcs = [stats(part_ref, me)] + [stats(rrecv_ref, k)
                                        for k in range(N_DEV - 1)]
        acc_srcs = [part_ref[me, :, :CH, :]] + [
            rrecv_ref[k, :, :CH, :] for k in range(N_DEV - 1)
        ]

        m_g = srcs[0][0]
        for m_, _ in srcs[1:]:
            m_g = jnp.maximum(m_g, m_)
        e_srcs = [jnp.exp(m_ - m_g) for m_, _ in srcs]
        l_tot = e_srcs[0] * srcs[0][1]
        for e_, (_, l_) in zip(e_srcs[1:], srcs[1:]):
            l_tot = l_tot + e_ * l_
        inv_l = 1.0 / l_tot

        for h in range(HQ):
            cols = slice(h * DH, (h + 1) * DH)
            num = (acc_srcs[0][:, :, cols].astype(jnp.float32)
                   * e_srcs[0][:, h, :][:, :, None])
            for s_ in range(1, N_DEV):
                num = num + (acc_srcs[s_][:, :, cols].astype(jnp.float32)
                             * e_srcs[s_][:, h, :][:, :, None])
            ag_ref[me, :, :, cols] = (num * inv_l[:, h, :][:, :, None]).astype(WIRE)

        ag_sends = []
        for t in range(1, N_DEV):
            tgt = lax.rem(me + t, N_DEV)
            rdma = pltpu.make_async_remote_copy(
                src_ref=ag_ref.at[me],
                dst_ref=ag_ref.at[me],
                send_sem=ag_send_sems.at[t - 1],
                recv_sem=ag_recv_sems.at[N_DEV - 1 - t],
                device_id=(tgt,), device_id_type=pl.DeviceIdType.MESH,
            )
            rdma.start()
            ag_sends.append(rdma)

        wo_w = wo_ref[...].astype(WIRE)
        for b in range(B):
            out_ref[b, pl.ds(me * CH, CH), :] = lax.dot(
                ag_ref[me, b], wo_w, preferred_element_type=jnp.float32)

        for k in range(N_DEV - 1):
            recv = pltpu.make_async_remote_copy(
                src_ref=ag_ref.at[0], dst_ref=ag_ref.at[me],
                send_sem=ag_send_sems.at[0],
                recv_sem=ag_recv_sems.at[k],
                device_id=(me,), device_id_type=pl.DeviceIdType.MESH,
            )
            recv.wait_recv()
            src = lax.rem(me + k + 1, N_DEV)
            for b in range(B):
                out_ref[b, pl.ds(src * CH, CH), :] = lax.dot(
                    ag_ref[src, b], wo_w, preferred_element_type=jnp.float32)

        for c, rdma in rs_sends:
            @pl.when(c != me)
            def _():
                rdma.wait_send()
        for rdma in ag_sends:
            rdma.wait_send()

    return pl.pallas_call(
        body,
        out_shape=jax.ShapeDtypeStruct((B, SQ, DM), jnp.float32),
        in_specs=[pl.BlockSpec(memory_space=pltpu.VMEM)] * 5,
        out_specs=pl.BlockSpec(memory_space=pltpu.VMEM),
        scratch_shapes=[
            pltpu.VMEM((N_DEV, B, PROWS, DQ), WIRE),
            pltpu.VMEM((N_DEV - 1, B, PROWS, DQ), WIRE),
            pltpu.VMEM((N_DEV, B, CH, DQ), WIRE),
            pltpu.SemaphoreType.DMA((B, N_DEV)),
            pltpu.SemaphoreType.DMA((B, N_DEV - 1)),
            pltpu.SemaphoreType.DMA((N_DEV - 1,)),
            pltpu.SemaphoreType.DMA((N_DEV - 1,)),
        ],
        compiler_params=pltpu.CompilerParams(collective_id=0),
    )(x, Wq, K_ext, V_ext, Wo)


# device time: 45790 ns/iter; 4.9042x vs baseline; 1.0325x over previous
import jax
import jax.numpy as jnp
from jax import lax
from jax.experimental import pallas as pl
from jax.experimental.pallas import tpu as pltpu

N_DEV = 8
B = 2
SQ = 512
DM = 768
HQ = 8
DH = 64
DQ = HQ * DH
CH = SQ // N_DEV
WIRE = jnp.bfloat16
PROWS = CH + 2 * HQ


def kernel(x, Wq, K_ext, V_ext, Wo):
    def body(x_ref, wq_ref, k_ref, v_ref, wo_ref, out_ref,
             part_ref,
             rrecv_ref,
             ag_ref,
             rs_send_sems,
             rs_recv_sems,
             ag_send_sems, ag_recv_sems):
        me = lax.axis_index("i")

        barrier_sem = pltpu.get_barrier_semaphore()
        for d in range(N_DEV):
            @pl.when(d != me)
            def _():
                pl.semaphore_signal(
                    barrier_sem, inc=1,
                    device_id=(d,), device_id_type=pl.DeviceIdType.MESH,
                )

        wq_w = wq_ref[...].astype(WIRE)
        rs_sends = []
        for b in range(B):
            x_b = x_ref[b].astype(WIRE)
            k_b = k_ref[b].astype(WIRE)
            v_b = v_ref[b].astype(WIRE)
            q_b = lax.dot(x_b, wq_w,
                          preferred_element_type=jnp.float32)
            q_b = q_b.astype(WIRE)
            for h in range(HQ):
                cols = slice(h * DH, (h + 1) * DH)
                for r in range(4):
                    lo = slice(r * 64, r * 64 + 64)
                    hi = slice((r + 4) * 64, (r + 4) * 64 + 64)
                    q_r = jnp.concatenate(
                        [q_b[lo, cols], q_b[hi, cols]], axis=0)
                    k_r = jnp.concatenate(
                        [k_b[lo, h, :], k_b[hi, h, :]], axis=0)
                    v_r = jnp.concatenate(
                        [v_b[lo, h, :], v_b[hi, h, :]], axis=0)
                    s = lax.dot_general(
                        q_r, k_r, (((1,), (1,)), ((), ())),
                        preferred_element_type=jnp.float32,
                    ) * 0.125
                    m_r = jnp.max(s, axis=1)
                    w = jnp.exp(s - m_r[:, None]).astype(WIRE)
                    l_r = jnp.sum(w.astype(jnp.float32), axis=1)
                    acc_r = lax.dot_general(
                        w, v_r, (((1,), (0,)), ((), ())),
                        preferred_element_type=jnp.float32,
                    ).astype(WIRE)
                    for half, qb in ((0, r), (1, r + 4)):
                        rows = slice(half * 64, half * 64 + 64)
                        part_ref[qb, b, :CH, cols] = acc_r[rows, :]
                        part_ref[qb, b, CH + h, :CH] = m_r[rows].astype(WIRE)
                        part_ref[qb, b, CH + HQ + h, :CH] = l_r[rows].astype(WIRE)
            if b == 0:
                pl.semaphore_wait(barrier_sem, N_DEV - 1)
            for c in range(N_DEV):
                slot = lax.rem(me - c - 1 + 2 * N_DEV, N_DEV)
                rdma = pltpu.make_async_remote_copy(
                    src_ref=part_ref.at[c, b],
                    dst_ref=rrecv_ref.at[slot, b],
                    send_sem=rs_send_sems.at[b, c],
                    recv_sem=rs_recv_sems.at[b, slot],
                    device_id=(c,), device_id_type=pl.DeviceIdType.MESH,
                )

                @pl.when(c != me)
                def _():
                    rdma.start()

                rs_sends.append((c, rdma))

        for b in range(B):
            for k in range(N_DEV - 1):
                recv = pltpu.make_async_remote_copy(
                    src_ref=part_ref.at[0, b], dst_ref=rrecv_ref.at[k, b],
                    send_sem=rs_send_sems.at[b, 0],
                    recv_sem=rs_recv_sems.at[b, k],
                    device_id=(me,), device_id_type=pl.DeviceIdType.MESH,
                )
                recv.wait_recv()

        def stats(ref, idx):
            m_ = ref[idx, :, CH:CH + HQ, :CH].astype(jnp.float32)
            l_ = ref[idx, :, CH + HQ:CH + 2 * HQ, :CH].astype(jnp.float32)
            return m_, l_

        srcs = [stats(part_ref, me)] + [stats(rrecv_ref, k)
                                        for k in range(N_DEV - 1)]
        acc_srcs = [part_ref[me, :, :CH, :]] + [
            rrecv_ref[k, :, :CH, :] for k in range(N_DEV - 1)
        ]

        m_g = srcs[0][0]
        for m_, _ in srcs[1:]:
            m_g = jnp.maximum(m_g, m_)
        e_srcs = [jnp.exp(m_ - m_g) for m_, _ in srcs]
        l_tot = e_srcs[0] * srcs[0][1]
        for e_, (_, l_) in zip(e_srcs[1:], srcs[1:]):
            l_tot = l_tot + e_ * l_
        inv_l = 1.0 / l_tot

        for h in range(HQ):
            cols = slice(h * DH, (h + 1) * DH)
            num = (acc_srcs[0][:, :, cols].astype(jnp.float32)
                   * e_srcs[0][:, h, :][:, :, None])
            for s_ in range(1, N_DEV):
                num = num + (acc_srcs[s_][:, :, cols].astype(jnp.float32)
                             * e_srcs[s_][:, h, :][:, :, None])
            ag_ref[me, :, :, cols] = (num * inv_l[:, h, :][:, :, None]).astype(WIRE)

        ag_sends = []
        for t in range(1, N_DEV):
            tgt = lax.rem(me + t, N_DEV)
            rdma = pltpu.make_async_remote_copy(
                src_ref=ag_ref.at[me],
                dst_ref=ag_ref.at[me],
                send_sem=ag_send_sems.at[t - 1],
                recv_sem=ag_recv_sems.at[N_DEV - 1 - t],
                device_id=(tgt,), device_id_type=pl.DeviceIdType.MESH,
            )
            rdma.start()
            ag_sends.append(rdma)

        wo_w = wo_ref[...].astype(WIRE)
        for b in range(B):
            out_ref[b, pl.ds(me * CH, CH), :] = lax.dot(
                ag_ref[me, b], wo_w, preferred_element_type=jnp.float32)

        for k in range(N_DEV - 1):
            recv = pltpu.make_async_remote_copy(
                src_ref=ag_ref.at[0], dst_ref=ag_ref.at[me],
                send_sem=ag_send_sems.at[0],
                recv_sem=ag_recv_sems.at[k],
                device_id=(me,), device_id_type=pl.DeviceIdType.MESH,
            )
            recv.wait_recv()
            src = lax.rem(me + k + 1, N_DEV)
            for b in range(B):
                out_ref[b, pl.ds(src * CH, CH), :] = lax.dot(
                    ag_ref[src, b], wo_w, preferred_element_type=jnp.float32)

        for c, rdma in rs_sends:
            @pl.when(c != me)
            def _():
                rdma.wait_send()
        for rdma in ag_sends:
            rdma.wait_send()

    return pl.pallas_call(
        body,
        out_shape=jax.ShapeDtypeStruct((B, SQ, DM), jnp.float32),
        in_specs=[pl.BlockSpec(memory_space=pltpu.VMEM)] * 5,
        out_specs=pl.BlockSpec(memory_space=pltpu.VMEM),
        scratch_shapes=[
            pltpu.VMEM((N_DEV, B, PROWS, DQ), WIRE),
            pltpu.VMEM((N_DEV - 1, B, PROWS, DQ), WIRE),
            pltpu.VMEM((N_DEV, B, CH, DQ), WIRE),
            pltpu.SemaphoreType.DMA((B, N_DEV)),
            pltpu.SemaphoreType.DMA((B, N_DEV - 1)),
            pltpu.SemaphoreType.DMA((N_DEV - 1,)),
            pltpu.SemaphoreType.DMA((N_DEV - 1,)),
        ],
        compiler_params=pltpu.CompilerParams(collective_id=0),
    )(x, Wq, K_ext, V_ext, Wo)


# device time: 41215 ns/iter; 5.4486x vs baseline; 1.1110x over previous
import jax
import jax.numpy as jnp
from jax import lax
from jax.experimental import pallas as pl
from jax.experimental.pallas import tpu as pltpu

N_DEV = 8
B = 2
SQ = 512
DM = 768
HQ = 8
DH = 64
DQ = HQ * DH
CH = SQ // N_DEV
WIRE = jnp.bfloat16
PROWS = CH + 2 * HQ


def kernel(x, Wq, K_ext, V_ext, Wo):
    def body(x_ref, wq_ref, k_ref, v_ref, wo_ref, out_ref,
             part_ref,
             rrecv_ref,
             ag_ref,
             rs_send_sems,
             rs_recv_sems,
             ag_send_sems, ag_recv_sems):
        me = lax.axis_index("i")

        barrier_sem = pltpu.get_barrier_semaphore()
        for d in range(N_DEV):
            @pl.when(d != me)
            def _():
                pl.semaphore_signal(
                    barrier_sem, inc=1,
                    device_id=(d,), device_id_type=pl.DeviceIdType.MESH,
                )

        wq_w = wq_ref[...].astype(WIRE)
        rs_sends = []
        for b in range(B):
            x_b = x_ref[b].astype(WIRE)
            k_b = k_ref[b].astype(WIRE)
            v_b = v_ref[b].astype(WIRE)
            q_b = lax.dot(x_b, wq_w,
                          preferred_element_type=jnp.float32)
            q_b = q_b.astype(WIRE)
            for h in range(HQ):
                cols = slice(h * DH, (h + 1) * DH)
                for r in range(4):
                    lo = slice(r * 64, r * 64 + 64)
                    hi = slice((r + 4) * 64, (r + 4) * 64 + 64)
                    q_r = jnp.concatenate(
                        [q_b[lo, cols], q_b[hi, cols]], axis=0)
                    k_r = jnp.concatenate(
                        [k_b[lo, h, :], k_b[hi, h, :]], axis=0)
                    v_r = jnp.concatenate(
                        [v_b[lo, h, :], v_b[hi, h, :]], axis=0)
                    s = lax.dot_general(
                        q_r, k_r, (((1,), (1,)), ((), ())),
                        preferred_element_type=jnp.float32,
                    ) * 0.125
                    m_r = jnp.max(s, axis=1)
                    w = jnp.exp(s - m_r[:, None]).astype(WIRE)
                    l_r = jnp.sum(w.astype(jnp.float32), axis=1)
                    acc_r = lax.dot_general(
                        w, v_r, (((1,), (0,)), ((), ())),
                        preferred_element_type=jnp.float32,
                    ).astype(WIRE)
                    for half, qb in ((0, r), (1, r + 4)):
                        rows = slice(half * 64, half * 64 + 64)
                        part_ref[qb, b, :CH, cols] = acc_r[rows, :]
                        part_ref[qb, b, CH + h, :CH] = m_r[rows].astype(WIRE)
                        part_ref[qb, b, CH + HQ + h, :CH] = l_r[rows].astype(WIRE)
            if b == 0:
                pl.semaphore_wait(barrier_sem, N_DEV - 1)
            for c in range(N_DEV):
                slot = lax.rem(me - c - 1 + 2 * N_DEV, N_DEV)
                rdma = pltpu.make_async_remote_copy(
                    src_ref=part_ref.at[c, b],
                    dst_ref=rrecv_ref.at[slot, b],
                    send_sem=rs_send_sems.at[b, c],
                    recv_sem=rs_recv_sems.at[b, slot],
                    device_id=(c,), device_id_type=pl.DeviceIdType.MESH,
                )

                @pl.when(c != me)
                def _():
                    rdma.start()

                rs_sends.append((c, rdma))

        wo_w = wo_ref[...].astype(WIRE)
        ag_sends = []
        for b in range(B):
            for k in range(N_DEV - 1):
                recv = pltpu.make_async_remote_copy(
                    src_ref=part_ref.at[0, b], dst_ref=rrecv_ref.at[k, b],
                    send_sem=rs_send_sems.at[b, 0],
                    recv_sem=rs_recv_sems.at[b, k],
                    device_id=(me,), device_id_type=pl.DeviceIdType.MESH,
                )
                recv.wait_recv()

            def stats(ref, idx):
                m_ = ref[idx, b, CH:CH + HQ, :CH].astype(jnp.float32)
                l_ = ref[idx, b, CH + HQ:CH + 2 * HQ, :CH].astype(jnp.float32)
                return m_, l_

            srcs = [stats(part_ref, me)] + [stats(rrecv_ref, k)
                                            for k in range(N_DEV - 1)]
            acc_srcs = [part_ref[me, b, :CH, :]] + [
                rrecv_ref[k, b, :CH, :] for k in range(N_DEV - 1)
            ]

            m_g = srcs[0][0]
            for m_, _ in srcs[1:]:
                m_g = jnp.maximum(m_g, m_)
            e_srcs = [jnp.exp(m_ - m_g) for m_, _ in srcs]
            l_tot = e_srcs[0] * srcs[0][1]
            for e_, (_, l_) in zip(e_srcs[1:], srcs[1:]):
                l_tot = l_tot + e_ * l_
            inv_l = 1.0 / l_tot

            for h in range(HQ):
                cols = slice(h * DH, (h + 1) * DH)
                num = (acc_srcs[0][:, cols].astype(jnp.float32)
                       * e_srcs[0][h][:, None])
                for s_ in range(1, N_DEV):
                    num = num + (acc_srcs[s_][:, cols].astype(jnp.float32)
                                 * e_srcs[s_][h][:, None])
                ag_ref[me, b, :, cols] = (num * inv_l[h][:, None]).astype(WIRE)

            for t in range(1, N_DEV):
                tgt = lax.rem(me + t, N_DEV)
                rdma = pltpu.make_async_remote_copy(
                    src_ref=ag_ref.at[me, b],
                    dst_ref=ag_ref.at[me, b],
                    send_sem=ag_send_sems.at[b, t - 1],
                    recv_sem=ag_recv_sems.at[b, N_DEV - 1 - t],
                    device_id=(tgt,), device_id_type=pl.DeviceIdType.MESH,
                )
                rdma.start()
                ag_sends.append(rdma)

            out_ref[b, pl.ds(me * CH, CH), :] = lax.dot(
                ag_ref[me, b], wo_w, preferred_element_type=jnp.float32)

        for b in range(B):
            for k in range(N_DEV - 1):
                recv = pltpu.make_async_remote_copy(
                    src_ref=ag_ref.at[0, b], dst_ref=ag_ref.at[me, b],
                    send_sem=ag_send_sems.at[b, 0],
                    recv_sem=ag_recv_sems.at[b, k],
                    device_id=(me,), device_id_type=pl.DeviceIdType.MESH,
                )
                recv.wait_recv()
                src = lax.rem(me + k + 1, N_DEV)
                out_ref[b, pl.ds(src * CH, CH), :] = lax.dot(
                    ag_ref[src, b], wo_w, preferred_element_type=jnp.float32)

        for c, rdma in rs_sends:
            @pl.when(c != me)
            def _():
                rdma.wait_send()
        for rdma in ag_sends:
            rdma.wait_send()

    return pl.pallas_call(
        body,
        out_shape=jax.ShapeDtypeStruct((B, SQ, DM), jnp.float32),
        in_specs=[pl.BlockSpec(memory_space=pltpu.VMEM)] * 5,
        out_specs=pl.BlockSpec(memory_space=pltpu.VMEM),
        scratch_shapes=[
            pltpu.VMEM((N_DEV, B, PROWS, DQ), WIRE),
            pltpu.VMEM((N_DEV - 1, B, PROWS, DQ), WIRE),
            pltpu.VMEM((N_DEV, B, CH, DQ), WIRE),
            pltpu.SemaphoreType.DMA((B, N_DEV)),
            pltpu.SemaphoreType.DMA((B, N_DEV - 1)),
            pltpu.SemaphoreType.DMA((B, N_DEV - 1)),
            pltpu.SemaphoreType.DMA((B, N_DEV - 1)),
        ],
        compiler_params=pltpu.CompilerParams(collective_id=0),
    )(x, Wq, K_ext, V_ext, Wo)


# device time: 41049 ns/iter; 5.4706x vs baseline; 1.0040x over previous
import jax
import jax.numpy as jnp
from jax import lax
from jax.experimental import pallas as pl
from jax.experimental.pallas import tpu as pltpu

N_DEV = 8
B = 2
SQ = 512
DM = 768
HQ = 8
DH = 64
DQ = HQ * DH
CH = SQ // N_DEV
WIRE = jnp.bfloat16
PROWS = CH + 2


def kernel(x, Wq, K_ext, V_ext, Wo):
    def body(x_ref, wq_ref, k_ref, v_ref, wo_ref, out_ref,
             part_ref,
             rrecv_ref,
             ag_ref,
             accrun_ref,
             rs_send_sems,
             rs_recv_sems,
             ag_send_sems, ag_recv_sems):
        me = lax.axis_index("i")

        barrier_sem = pltpu.get_barrier_semaphore()
        for d in range(N_DEV):
            @pl.when(d != me)
            def _():
                pl.semaphore_signal(
                    barrier_sem, inc=1,
                    device_id=(d,), device_id_type=pl.DeviceIdType.MESH,
                )

        wq_w = wq_ref[...].astype(WIRE)
        rs_sends = []
        for b in range(B):
            x_b = x_ref[b].astype(WIRE)
            k_b = k_ref[b].astype(WIRE)
            v_b = v_ref[b].astype(WIRE)
            q_b = lax.dot(x_b, wq_w,
                          preferred_element_type=jnp.float32)
            q_b = q_b.astype(WIRE)
            for h in range(HQ):
                cols = slice(h * DH, (h + 1) * DH)
                for r in range(4):
                    lo = slice(r * 64, r * 64 + 64)
                    hi = slice((r + 4) * 64, (r + 4) * 64 + 64)
                    q_r = jnp.concatenate(
                        [q_b[lo, cols], q_b[hi, cols]], axis=0)
                    k_r = jnp.concatenate(
                        [k_b[lo, h, :], k_b[hi, h, :]], axis=0)
                    v_r = jnp.concatenate(
                        [v_b[lo, h, :], v_b[hi, h, :]], axis=0)
                    s = lax.dot_general(
                        q_r, k_r, (((1,), (1,)), ((), ())),
                        preferred_element_type=jnp.float32,
                    ) * 0.125
                    m_r = jnp.max(s, axis=1)
                    w = jnp.exp(s - m_r[:, None]).astype(WIRE)
                    l_r = jnp.sum(w.astype(jnp.float32), axis=1)
                    acc_r = lax.dot_general(
                        w, v_r, (((1,), (0,)), ((), ())),
                        preferred_element_type=jnp.float32,
                    ).astype(WIRE)
                    for half, qb in ((0, r), (1, r + 4)):
                        rows = slice(half * 64, half * 64 + 64)
                        part_ref[qb, b, :CH, cols] = acc_r[rows, :]
                        part_ref[qb, b, CH, cols] = m_r[rows].astype(WIRE)
                        part_ref[qb, b, CH + 1, cols] = l_r[rows].astype(WIRE)
            if b == 0:
                pl.semaphore_wait(barrier_sem, N_DEV - 1)
            for c in range(N_DEV):
                slot = lax.rem(me - c - 1 + 2 * N_DEV, N_DEV)
                rdma = pltpu.make_async_remote_copy(
                    src_ref=part_ref.at[c, b],
                    dst_ref=rrecv_ref.at[slot, b],
                    send_sem=rs_send_sems.at[b, c],
                    recv_sem=rs_recv_sems.at[b, slot],
                    device_id=(c,), device_id_type=pl.DeviceIdType.MESH,
                )

                @pl.when(c != me)
                def _():
                    rdma.start()

                rs_sends.append((c, rdma))

        wo_w = wo_ref[...].astype(WIRE)
        ag_sends = []
        for b in range(B):
            accrun_ref[b] = part_ref[me, b, :CH, :].astype(jnp.float32)
            m_run = part_ref[me, b, CH, :].astype(jnp.float32)
            l_run = part_ref[me, b, CH + 1, :].astype(jnp.float32)
            for k in range(N_DEV - 1):
                recv = pltpu.make_async_remote_copy(
                    src_ref=part_ref.at[0, b], dst_ref=rrecv_ref.at[k, b],
                    send_sem=rs_send_sems.at[b, 0],
                    recv_sem=rs_recv_sems.at[b, k],
                    device_id=(me,), device_id_type=pl.DeviceIdType.MESH,
                )
                recv.wait_recv()
                m_k = rrecv_ref[k, b, CH, :].astype(jnp.float32)
                l_k = rrecv_ref[k, b, CH + 1, :].astype(jnp.float32)
                m_new = jnp.maximum(m_run, m_k)
                a_s = jnp.exp(m_run - m_new)
                e_s = jnp.exp(m_k - m_new)
                l_run = a_s * l_run + e_s * l_k
                m_run = m_new
                for h in range(HQ):
                    cols = slice(h * DH, (h + 1) * DH)
                    accrun_ref[b, :, cols] = (
                        accrun_ref[b, :, cols] * a_s[cols][:, None]
                        + rrecv_ref[k, b, :CH, cols].astype(jnp.float32)
                        * e_s[cols][:, None])

            inv_l = 1.0 / l_run
            for h in range(HQ):
                cols = slice(h * DH, (h + 1) * DH)
                ag_ref[me, b, :, cols] = (
                    accrun_ref[b, :, cols] * inv_l[cols][:, None]).astype(WIRE)

            for t in range(1, N_DEV):
                tgt = lax.rem(me + t, N_DEV)
                rdma = pltpu.make_async_remote_copy(
                    src_ref=ag_ref.at[me, b],
                    dst_ref=ag_ref.at[me, b],
                    send_sem=ag_send_sems.at[b, t - 1],
                    recv_sem=ag_recv_sems.at[b, N_DEV - 1 - t],
                    device_id=(tgt,), device_id_type=pl.DeviceIdType.MESH,
                )
                rdma.start()
                ag_sends.append(rdma)

            out_ref[b, pl.ds(me * CH, CH), :] = lax.dot(
                ag_ref[me, b], wo_w, preferred_element_type=jnp.float32)

        for b in range(B):
            for k in range(N_DEV - 1):
                recv = pltpu.make_async_remote_copy(
                    src_ref=ag_ref.at[0, b], dst_ref=ag_ref.at[me, b],
                    send_sem=ag_send_sems.at[b, 0],
                    recv_sem=ag_recv_sems.at[b, k],
                    device_id=(me,), device_id_type=pl.DeviceIdType.MESH,
                )
                recv.wait_recv()
                src = lax.rem(me + k + 1, N_DEV)
                out_ref[b, pl.ds(src * CH, CH), :] = lax.dot(
                    ag_ref[src, b], wo_w, preferred_element_type=jnp.float32)

        for c, rdma in rs_sends:
            @pl.when(c != me)
            def _():
                rdma.wait_send()
        for rdma in ag_sends:
            rdma.wait_send()

    return pl.pallas_call(
        body,
        out_shape=jax.ShapeDtypeStruct((B, SQ, DM), jnp.float32),
        in_specs=[pl.BlockSpec(memory_space=pltpu.VMEM)] * 5,
        out_specs=pl.BlockSpec(memory_space=pltpu.VMEM),
        scratch_shapes=[
            pltpu.VMEM((N_DEV, B, PROWS, DQ), WIRE),
            pltpu.VMEM((N_DEV - 1, B, PROWS, DQ), WIRE),
            pltpu.VMEM((N_DEV, B, CH, DQ), WIRE),
            pltpu.VMEM((B, CH, DQ), jnp.float32),
            pltpu.SemaphoreType.DMA((B, N_DEV)),
            pltpu.SemaphoreType.DMA((B, N_DEV - 1)),
            pltpu.SemaphoreType.DMA((B, N_DEV - 1)),
            pltpu.SemaphoreType.DMA((B, N_DEV - 1)),
        ],
        compiler_params=pltpu.CompilerParams(collective_id=0),
    )(x, Wq, K_ext, V_ext, Wo)
